# baseline (speedup 1.0000x reference)
import functools

import jax
import jax.numpy as jnp
from jax import lax
from jax.experimental import pallas as pl
from jax.experimental.pallas import tpu as pltpu
from jax.experimental.pallas import tpu_sc as plsc

N = 10000
E = 320000
F = 128
NW = 32
CB = 100
DEG_N = 10240
DNCH = E // NW // CB

FH = F // 2
CBM = 125
GRP = 16
NGRP = E // 16 // (GRP * CBM)
NP = 10240
RPT = NP // 16

_mesh = plsc.VectorSubcoreMesh(core_axis_name="c", subcore_axis_name="s")


@functools.partial(
    pl.kernel,
    mesh=_mesh,
    out_type=jax.ShapeDtypeStruct((2, DEG_N), jnp.float32),
    scratch_types=[
        pltpu.VMEM((DNCH, CB), jnp.int32),
        pltpu.VMEM((128,), jnp.float32),
        pltpu.VMEM((DEG_N // 16,), jnp.float32),
        pltpu.VMEM_SHARED((DEG_N,), jnp.float32),
    ],
)
def _deg_kernel(dst_hbm, out_hbm, idx_v, ones_v, zero_v, acc_sh):
    c = lax.axis_index("c")
    s = lax.axis_index("s")
    w = s * 2 + c
    for k in range(8):
        ones_v[pl.ds(k * 16, 16)] = jnp.ones((16,), jnp.float32)
    for k in range(DEG_N // 256):
        zero_v[pl.ds(k * 16, 16)] = jnp.zeros((16,), jnp.float32)
    pltpu.sync_copy(zero_v, acc_sh.at[pl.ds(s * (DEG_N // 16), DEG_N // 16)])
    plsc.subcore_barrier()
    pltpu.sync_copy(dst_hbm.at[w], idx_v)

    def chunk(j, carry):
        pltpu.sync_copy(ones_v.at[pl.ds(0, CB)], acc_sh.at[idx_v.at[j]],
                        add=True)
        return carry

    lax.fori_loop(0, DNCH, chunk, 0)
    plsc.subcore_barrier()
    pltpu.sync_copy(acc_sh.at[pl.ds(s * (DEG_N // 16), DEG_N // 16)],
                    out_hbm.at[c, pl.ds(s * (DEG_N // 16), DEG_N // 16)])


@functools.partial(
    pl.kernel,
    mesh=_mesh,
    out_type=jax.ShapeDtypeStruct((2, NP, FH), jnp.float32),
    compiler_params=pltpu.CompilerParams(use_tc_tiling_on_sc=False),
    scratch_types=[
        pltpu.VMEM((GRP, CBM), jnp.int32),
        pltpu.VMEM((GRP, CBM), jnp.int32),
        pltpu.VMEM((GRP, CBM), jnp.int32),
        pltpu.VMEM((GRP, CBM), jnp.int32),
        pltpu.VMEM((CBM, FH), jnp.float32),
        pltpu.VMEM((CBM, FH), jnp.float32),
        pltpu.VMEM_SHARED((NP, FH), jnp.float32),
        pltpu.SemaphoreType.DMA,
        pltpu.SemaphoreType.DMA,
        pltpu.SemaphoreType.DMA,
        pltpu.SemaphoreType.DMA,
    ],
)
def _msg_kernel(hs_hbm, src_hbm, dst_hbm, out_hbm,
                src_a, dst_a, src_b, dst_b, rows0, rows1, acc_sh,
                sem0, sem1, sem_ia, sem_ib):
    c = lax.axis_index("c")
    s = lax.axis_index("s")
    hs_c = hs_hbm.at[c]

    def zrow(i, carry):
        for k in range(FH // 16):
            rows0[i, pl.ds(k * 16, 16)] = jnp.zeros((16,), jnp.float32)
        return carry

    lax.fori_loop(0, CBM, zrow, 0)
    for k in range(RPT // CBM):
        pltpu.sync_copy(rows0, acc_sh.at[pl.ds(s * RPT + k * CBM, CBM)])
    pltpu.sync_copy(rows0.at[pl.ds(0, RPT % CBM)],
                    acc_sh.at[pl.ds(s * RPT + (RPT // CBM) * CBM, RPT % CBM)])
    plsc.subcore_barrier()

    bufs = (rows0, rows1)
    sems = (sem0, sem1)

    def run_group(sv, dv):
        pltpu.async_copy(hs_c.at[sv.at[0]], rows0, sem0)
        for j in range(GRP):
            if j + 1 < GRP:
                pltpu.async_copy(hs_c.at[sv.at[j + 1]],
                                 bufs[(j + 1) % 2], sems[(j + 1) % 2])
            pltpu.make_async_copy(hs_c.at[sv.at[j]],
                                  bufs[j % 2], sems[j % 2]).wait()
            pltpu.sync_copy(bufs[j % 2], acc_sh.at[dv.at[j]], add=True)

    pltpu.async_copy(src_hbm.at[s * NGRP], src_a, sem_ia)
    pltpu.async_copy(dst_hbm.at[s * NGRP], dst_a, sem_ia)

    def pair(i, carry):
        g = 2 * i
        pltpu.make_async_copy(src_hbm.at[s * NGRP + g], src_a, sem_ia).wait()
        pltpu.make_async_copy(dst_hbm.at[s * NGRP + g], dst_a, sem_ia).wait()
        pltpu.async_copy(src_hbm.at[s * NGRP + g + 1], src_b, sem_ib)
        pltpu.async_copy(dst_hbm.at[s * NGRP + g + 1], dst_b, sem_ib)
        run_group(src_a, dst_a)
        pltpu.make_async_copy(src_hbm.at[s * NGRP + g + 1], src_b,
                              sem_ib).wait()
        pltpu.make_async_copy(dst_hbm.at[s * NGRP + g + 1], dst_b,
                              sem_ib).wait()

        @pl.when(g + 2 < NGRP)
        def _():
            pltpu.async_copy(src_hbm.at[s * NGRP + g + 2], src_a, sem_ia)
            pltpu.async_copy(dst_hbm.at[s * NGRP + g + 2], dst_a, sem_ia)

        run_group(src_b, dst_b)
        return carry

    lax.fori_loop(0, NGRP // 2, pair, 0)
    plsc.subcore_barrier()
    pltpu.sync_copy(acc_sh.at[pl.ds(s * RPT, RPT)],
                    out_hbm.at[c, pl.ds(s * RPT, RPT)])


RB = 2000
NRB = N // RB


def _mm1_body(x_ref, w_ref, b_ref, d_ref, o_ref):
    h = jnp.dot(x_ref[...], w_ref[...], preferred_element_type=jnp.float32)
    h = h + b_ref[...]
    o_ref[0] = h[:, :FH] * d_ref[...]
    o_ref[1] = h[:, FH:] * d_ref[...]


def _relu_halves(p_ref, hs_ref, d_ref):
    tl = jnp.maximum((p_ref[0] + hs_ref[0]) * d_ref[...], 0.0)
    th = jnp.maximum((p_ref[1] + hs_ref[1]) * d_ref[...], 0.0)
    return tl, th


def _mm2_body(p_ref, hs_ref, d_ref, w_ref, b_ref, o_ref):
    tl, th = _relu_halves(p_ref, hs_ref, d_ref)
    h = jnp.dot(tl, w_ref[:FH], preferred_element_type=jnp.float32)
    h = h + jnp.dot(th, w_ref[FH:], preferred_element_type=jnp.float32)
    h = h + b_ref[...]
    o_ref[0] = h[:, :FH] * d_ref[...]
    o_ref[1] = h[:, FH:] * d_ref[...]


def _finlog_body(p_ref, hs_ref, d_ref, wh_ref, bh_ref, o_ref):
    tl, th = _relu_halves(p_ref, hs_ref, d_ref)
    logits = jnp.dot(tl, wh_ref[:FH], preferred_element_type=jnp.float32)
    logits = logits + jnp.dot(th, wh_ref[FH:],
                              preferred_element_type=jnp.float32)
    o_ref[...] = logits + bh_ref[...]


def _smax_body(l_ref, o_ref):
    logits = l_ref[...]
    m = jnp.max(logits)
    e = jnp.exp(logits - m)
    o_ref[...] = e / jnp.sum(e)


def kernel(x, ei, W1, b1, W2, b2, Wh, bh):
    ei = ei.astype(jnp.int32)
    src = ei[0].reshape(16 * NGRP, GRP, CBM)
    dst = ei[1].reshape(16 * NGRP, GRP, CBM)
    dst_deg = ei[1].reshape(NW, DNCH, CB)

    degp = _deg_kernel(dst_deg)
    deg = degp[0, :N] + degp[1, :N] + 1.0
    dinvb = jnp.broadcast_to(lax.rsqrt(deg)[:, None], (N, FH))

    full = lambda shape: pl.BlockSpec(shape, lambda i: (0,) * len(shape))
    rows2 = pl.BlockSpec((RB, F), lambda i: (i, 0))
    rows_h = pl.BlockSpec((RB, FH), lambda i: (i, 0))
    rows3 = pl.BlockSpec((2, RB, FH), lambda i: (0, i, 0))

    hs1 = pl.pallas_call(
        _mm1_body,
        grid=(NRB,),
        in_specs=[rows2, full((F, F)), full((1, F)), rows_h],
        out_specs=rows3,
        out_shape=jax.ShapeDtypeStruct((2, N, FH), jnp.float32),
    )(x, W1, b1.reshape(1, F), dinvb)

    p1 = _msg_kernel(hs1, src, dst)

    hs2 = pl.pallas_call(
        _mm2_body,
        grid=(NRB,),
        in_specs=[rows3, rows3, rows_h, full((F, F)), full((1, F))],
        out_specs=rows3,
        out_shape=jax.ShapeDtypeStruct((2, N, FH), jnp.float32),
    )(p1, hs1, dinvb, W2, b2.reshape(1, F))

    p2 = _msg_kernel(hs2, src, dst)

    logits = pl.pallas_call(
        _finlog_body,
        grid=(NRB,),
        in_specs=[rows3, rows3, rows_h, full((F, 1)), full((1, 1))],
        out_specs=pl.BlockSpec((RB, 1), lambda i: (i, 0)),
        out_shape=jax.ShapeDtypeStruct((N, 1), jnp.float32),
    )(p2, hs2, dinvb, Wh, bh.reshape(1, 1))

    w = pl.pallas_call(
        _smax_body,
        out_shape=jax.ShapeDtypeStruct((N, 1), jnp.float32),
    )(logits)

    return w[:, 0]


# speedup vs baseline: 1.0078x; 1.0078x over previous
import functools

import jax
import jax.numpy as jnp
from jax import lax
from jax.experimental import pallas as pl
from jax.experimental.pallas import tpu as pltpu
from jax.experimental.pallas import tpu_sc as plsc

N = 10000
E = 320000
F = 128
NW = 32
CB = 100
DEG_N = 10240
DNCH = E // NW // CB

FH = F // 2
CBM = 125
GRP = 16
NGRP = E // 16 // (GRP * CBM)
NP = 10240
RPT = NP // 16

_mesh = plsc.VectorSubcoreMesh(core_axis_name="c", subcore_axis_name="s")


@functools.partial(
    pl.kernel,
    mesh=_mesh,
    out_type=jax.ShapeDtypeStruct((2, DEG_N), jnp.float32),
    compiler_params=pltpu.CompilerParams(use_tc_tiling_on_sc=False),
    scratch_types=[
        pltpu.VMEM((DNCH, CB), jnp.int32),
        pltpu.VMEM((128,), jnp.float32),
        pltpu.VMEM((DEG_N // 16,), jnp.float32),
        pltpu.VMEM_SHARED((DEG_N,), jnp.float32),
    ],
)
def _deg_kernel(dst_hbm, out_hbm, idx_v, ones_v, zero_v, acc_sh):
    c = lax.axis_index("c")
    s = lax.axis_index("s")
    w = NW + s * 2 + c
    for k in range(8):
        ones_v[pl.ds(k * 16, 16)] = jnp.ones((16,), jnp.float32)
    for k in range(DEG_N // 256):
        zero_v[pl.ds(k * 16, 16)] = jnp.zeros((16,), jnp.float32)
    pltpu.sync_copy(zero_v, acc_sh.at[pl.ds(s * (DEG_N // 16), DEG_N // 16)])
    plsc.subcore_barrier()
    pltpu.sync_copy(dst_hbm.at[w], idx_v)

    def chunk(j, carry):
        pltpu.sync_copy(ones_v.at[pl.ds(0, CB)], acc_sh.at[idx_v.at[j]],
                        add=True)
        return carry

    lax.fori_loop(0, DNCH, chunk, 0)
    plsc.subcore_barrier()
    pltpu.sync_copy(acc_sh.at[pl.ds(s * (DEG_N // 16), DEG_N // 16)],
                    out_hbm.at[c, pl.ds(s * (DEG_N // 16), DEG_N // 16)])


@functools.partial(
    pl.kernel,
    mesh=_mesh,
    out_type=jax.ShapeDtypeStruct((2, NP, FH), jnp.float32),
    compiler_params=pltpu.CompilerParams(use_tc_tiling_on_sc=False),
    scratch_types=[
        pltpu.VMEM((GRP, CBM), jnp.int32),
        pltpu.VMEM((GRP, CBM), jnp.int32),
        pltpu.VMEM((GRP, CBM), jnp.int32),
        pltpu.VMEM((GRP, CBM), jnp.int32),
        pltpu.VMEM((CBM, FH), jnp.float32),
        pltpu.VMEM((CBM, FH), jnp.float32),
        pltpu.VMEM_SHARED((NP, FH), jnp.float32),
        pltpu.SemaphoreType.DMA,
        pltpu.SemaphoreType.DMA,
        pltpu.SemaphoreType.DMA,
        pltpu.SemaphoreType.DMA,
    ],
)
def _msg_kernel(hs_hbm, ei_hbm, out_hbm,
                src_a, dst_a, src_b, dst_b, rows0, rows1, acc_sh,
                sem0, sem1, sem_ia, sem_ib):
    c = lax.axis_index("c")
    s = lax.axis_index("s")
    hs_c = hs_hbm.at[c]
    DOFF = 16 * NGRP

    def zrow(i, carry):
        for k in range(FH // 16):
            rows0[i, pl.ds(k * 16, 16)] = jnp.zeros((16,), jnp.float32)
        return carry

    lax.fori_loop(0, CBM, zrow, 0)
    for k in range(RPT // CBM):
        pltpu.sync_copy(rows0, acc_sh.at[pl.ds(s * RPT + k * CBM, CBM)])
    pltpu.sync_copy(rows0.at[pl.ds(0, RPT % CBM)],
                    acc_sh.at[pl.ds(s * RPT + (RPT // CBM) * CBM, RPT % CBM)])
    plsc.subcore_barrier()

    bufs = (rows0, rows1)
    sems = (sem0, sem1)

    def run_group(sv, dv):
        pltpu.async_copy(hs_c.at[sv.at[0]], rows0, sem0)
        for j in range(GRP):
            if j + 1 < GRP:
                pltpu.async_copy(hs_c.at[sv.at[j + 1]],
                                 bufs[(j + 1) % 2], sems[(j + 1) % 2])
            pltpu.make_async_copy(hs_c.at[sv.at[j]],
                                  bufs[j % 2], sems[j % 2]).wait()
            pltpu.sync_copy(bufs[j % 2], acc_sh.at[dv.at[j]], add=True)

    pltpu.async_copy(ei_hbm.at[s * NGRP], src_a, sem_ia)
    pltpu.async_copy(ei_hbm.at[DOFF + s * NGRP], dst_a, sem_ia)

    def pair(i, carry):
        g = 2 * i
        pltpu.make_async_copy(ei_hbm.at[s * NGRP + g], src_a, sem_ia).wait()
        pltpu.make_async_copy(ei_hbm.at[DOFF + s * NGRP + g], dst_a,
                              sem_ia).wait()
        pltpu.async_copy(ei_hbm.at[s * NGRP + g + 1], src_b, sem_ib)
        pltpu.async_copy(ei_hbm.at[DOFF + s * NGRP + g + 1], dst_b, sem_ib)
        run_group(src_a, dst_a)
        pltpu.make_async_copy(ei_hbm.at[s * NGRP + g + 1], src_b,
                              sem_ib).wait()
        pltpu.make_async_copy(ei_hbm.at[DOFF + s * NGRP + g + 1], dst_b,
                              sem_ib).wait()

        @pl.when(g + 2 < NGRP)
        def _():
            pltpu.async_copy(ei_hbm.at[s * NGRP + g + 2], src_a, sem_ia)
            pltpu.async_copy(ei_hbm.at[DOFF + s * NGRP + g + 2], dst_a,
                             sem_ia)

        run_group(src_b, dst_b)
        return carry

    lax.fori_loop(0, NGRP // 2, pair, 0)
    plsc.subcore_barrier()
    pltpu.sync_copy(acc_sh.at[pl.ds(s * RPT, RPT)],
                    out_hbm.at[c, pl.ds(s * RPT, RPT)])


RB = 2000
NRB = N // RB


def _mm1_body(x_ref, w_ref, b_ref, d_ref, o_ref):
    h = jnp.dot(x_ref[...], w_ref[...], preferred_element_type=jnp.float32)
    h = h + b_ref[...]
    o_ref[0] = h[:, :FH] * d_ref[...]
    o_ref[1] = h[:, FH:] * d_ref[...]


def _relu_halves(p_ref, hs_ref, d_ref):
    tl = jnp.maximum((p_ref[0] + hs_ref[0]) * d_ref[...], 0.0)
    th = jnp.maximum((p_ref[1] + hs_ref[1]) * d_ref[...], 0.0)
    return tl, th


def _mm2_body(p_ref, hs_ref, d_ref, w_ref, b_ref, o_ref):
    tl, th = _relu_halves(p_ref, hs_ref, d_ref)
    h = jnp.dot(tl, w_ref[:FH], preferred_element_type=jnp.float32)
    h = h + jnp.dot(th, w_ref[FH:], preferred_element_type=jnp.float32)
    h = h + b_ref[...]
    o_ref[0] = h[:, :FH] * d_ref[...]
    o_ref[1] = h[:, FH:] * d_ref[...]


def _finlog_body(p_ref, hs_ref, d_ref, wh_ref, bh_ref, o_ref):
    tl, th = _relu_halves(p_ref, hs_ref, d_ref)
    logits = jnp.dot(tl, wh_ref[:FH], preferred_element_type=jnp.float32)
    logits = logits + jnp.dot(th, wh_ref[FH:],
                              preferred_element_type=jnp.float32)
    o_ref[...] = logits + bh_ref[...]


def _smax_body(l_ref, o_ref):
    logits = l_ref[...]
    m = jnp.max(logits)
    e = jnp.exp(logits - m)
    o_ref[...] = e / jnp.sum(e)


def kernel(x, ei, W1, b1, W2, b2, Wh, bh):
    ei = ei.astype(jnp.int32)
    ei_msg = ei.reshape(2 * 16 * NGRP, GRP, CBM)
    ei_deg = ei.reshape(2 * NW, DNCH, CB)

    degp = _deg_kernel(ei_deg)
    deg = degp[0, :N] + degp[1, :N] + 1.0
    dinvb = jnp.broadcast_to(lax.rsqrt(deg)[:, None], (N, FH))

    full = lambda shape: pl.BlockSpec(shape, lambda i: (0,) * len(shape))
    rows2 = pl.BlockSpec((RB, F), lambda i: (i, 0))
    rows_h = pl.BlockSpec((RB, FH), lambda i: (i, 0))
    rows3 = pl.BlockSpec((2, RB, FH), lambda i: (0, i, 0))

    hs1 = pl.pallas_call(
        _mm1_body,
        grid=(NRB,),
        in_specs=[rows2, full((F, F)), full((1, F)), rows_h],
        out_specs=rows3,
        out_shape=jax.ShapeDtypeStruct((2, N, FH), jnp.float32),
    )(x, W1, b1.reshape(1, F), dinvb)

    p1 = _msg_kernel(hs1, ei_msg)

    hs2 = pl.pallas_call(
        _mm2_body,
        grid=(NRB,),
        in_specs=[rows3, rows3, rows_h, full((F, F)), full((1, F))],
        out_specs=rows3,
        out_shape=jax.ShapeDtypeStruct((2, N, FH), jnp.float32),
    )(p1, hs1, dinvb, W2, b2.reshape(1, F))

    p2 = _msg_kernel(hs2, ei_msg)

    logits = pl.pallas_call(
        _finlog_body,
        grid=(NRB,),
        in_specs=[rows3, rows3, rows_h, full((F, 1)), full((1, 1))],
        out_specs=pl.BlockSpec((RB, 1), lambda i: (i, 0)),
        out_shape=jax.ShapeDtypeStruct((N, 1), jnp.float32),
    )(p2, hs2, dinvb, Wh, bh.reshape(1, 1))

    w = pl.pallas_call(
        _smax_body,
        out_shape=jax.ShapeDtypeStruct((N, 1), jnp.float32),
    )(logits)

    return w[:, 0]


# speedup vs baseline: 1.0895x; 1.0811x over previous
import functools

import jax
import jax.numpy as jnp
from jax import lax
from jax.experimental import pallas as pl
from jax.experimental.pallas import tpu as pltpu
from jax.experimental.pallas import tpu_sc as plsc

N = 10000
E = 320000
F = 128
NW = 32
CB = 100
DEG_N = 10240
DNCH = E // NW // CB

FH = F // 2
CC = 200
CHG = 10
GEDG = CC * CHG
NGRP = E // 16 // GEDG
NP = 10240
RPT = NP // 16

_mesh = plsc.VectorSubcoreMesh(core_axis_name="c", subcore_axis_name="s")


@functools.partial(
    pl.kernel,
    mesh=_mesh,
    out_type=jax.ShapeDtypeStruct((2, DEG_N), jnp.float32),
    compiler_params=pltpu.CompilerParams(use_tc_tiling_on_sc=False),
    scratch_types=[
        pltpu.VMEM((DNCH, CB), jnp.int32),
        pltpu.VMEM((128,), jnp.float32),
        pltpu.VMEM((DEG_N // 16,), jnp.float32),
        pltpu.VMEM_SHARED((DEG_N,), jnp.float32),
    ],
)
def _deg_kernel(dst_hbm, out_hbm, idx_v, ones_v, zero_v, acc_sh):
    c = lax.axis_index("c")
    s = lax.axis_index("s")
    w = NW + s * 2 + c
    for k in range(8):
        ones_v[pl.ds(k * 16, 16)] = jnp.ones((16,), jnp.float32)
    for k in range(DEG_N // 256):
        zero_v[pl.ds(k * 16, 16)] = jnp.zeros((16,), jnp.float32)
    pltpu.sync_copy(zero_v, acc_sh.at[pl.ds(s * (DEG_N // 16), DEG_N // 16)])
    plsc.subcore_barrier()
    pltpu.sync_copy(dst_hbm.at[w], idx_v)

    def chunk(j, carry):
        pltpu.sync_copy(ones_v.at[pl.ds(0, CB)], acc_sh.at[idx_v.at[j]],
                        add=True)
        return carry

    lax.fori_loop(0, DNCH, chunk, 0)
    plsc.subcore_barrier()
    pltpu.sync_copy(acc_sh.at[pl.ds(s * (DEG_N // 16), DEG_N // 16)],
                    out_hbm.at[c, pl.ds(s * (DEG_N // 16), DEG_N // 16)])


@functools.partial(
    pl.kernel,
    mesh=_mesh,
    out_type=jax.ShapeDtypeStruct((2, NP, FH), jnp.float32),
    compiler_params=pltpu.CompilerParams(use_tc_tiling_on_sc=False),
    scratch_types=[
        pltpu.VMEM((GEDG,), jnp.int32),
        pltpu.VMEM((GEDG,), jnp.int32),
        pltpu.VMEM((GEDG,), jnp.int32),
        pltpu.VMEM((GEDG,), jnp.int32),
        pltpu.VMEM((CC, FH), jnp.float32),
        pltpu.VMEM((CC, FH), jnp.float32),
        pltpu.VMEM_SHARED((NP, FH), jnp.float32),
        pltpu.SemaphoreType.DMA,
        pltpu.SemaphoreType.DMA,
        pltpu.SemaphoreType.DMA,
        pltpu.SemaphoreType.DMA,
    ],
)
def _msg_kernel(hs_hbm, ei_hbm, out_hbm,
                src_a, dst_a, src_b, dst_b, rows0, rows1, acc_sh,
                sem0, sem1, sem_ia, sem_ib):
    c = lax.axis_index("c")
    s = lax.axis_index("s")
    hs_c = hs_hbm.at[c]

    def zrow(i, carry):
        for k in range(FH // 16):
            rows0[i, pl.ds(k * 16, 16)] = jnp.zeros((16,), jnp.float32)
        return carry

    lax.fori_loop(0, CC, zrow, 0)
    for k in range(RPT // CC):
        pltpu.sync_copy(rows0, acc_sh.at[pl.ds(s * RPT + k * CC, CC)])
    pltpu.sync_copy(rows0.at[pl.ds(0, RPT % CC)],
                    acc_sh.at[pl.ds(s * RPT + (RPT // CC) * CC, RPT % CC)])
    plsc.subcore_barrier()

    bufs = (rows0, rows1)
    sems = (sem0, sem1)

    def run_group(sv, dv):
        pltpu.async_copy(hs_c.at[sv.at[pl.ds(0, CC)]], rows0, sem0)
        for j in range(CHG):
            if j + 1 < CHG:
                pltpu.async_copy(hs_c.at[sv.at[pl.ds((j + 1) * CC, CC)]],
                                 bufs[(j + 1) % 2], sems[(j + 1) % 2])
            pltpu.make_async_copy(hs_c.at[sv.at[pl.ds(j * CC, CC)]],
                                  bufs[j % 2], sems[j % 2]).wait()
            pltpu.sync_copy(bufs[j % 2],
                            acc_sh.at[dv.at[pl.ds(j * CC, CC)]], add=True)

    def idx_fetch(g, sb, db, sem):
        base = (s * NGRP + g) * GEDG
        pltpu.async_copy(ei_hbm.at[pl.ds(base, GEDG)], sb, sem)
        pltpu.async_copy(ei_hbm.at[pl.ds(E + base, GEDG)], db, sem)

    def idx_wait(g, sb, db, sem):
        base = (s * NGRP + g) * GEDG
        pltpu.make_async_copy(ei_hbm.at[pl.ds(base, GEDG)], sb, sem).wait()
        pltpu.make_async_copy(ei_hbm.at[pl.ds(E + base, GEDG)], db,
                              sem).wait()

    idx_fetch(0, src_a, dst_a, sem_ia)

    def pair(i, carry):
        g = 2 * i
        idx_wait(g, src_a, dst_a, sem_ia)
        idx_fetch(g + 1, src_b, dst_b, sem_ib)
        run_group(src_a, dst_a)
        idx_wait(g + 1, src_b, dst_b, sem_ib)

        @pl.when(g + 2 < NGRP)
        def _():
            idx_fetch(g + 2, src_a, dst_a, sem_ia)

        run_group(src_b, dst_b)
        return carry

    lax.fori_loop(0, NGRP // 2, pair, 0)
    plsc.subcore_barrier()
    pltpu.sync_copy(acc_sh.at[pl.ds(s * RPT, RPT)],
                    out_hbm.at[c, pl.ds(s * RPT, RPT)])


RB = 2000
NRB = N // RB


def _mm1_body(x_ref, w_ref, b_ref, d_ref, o_ref):
    h = jnp.dot(x_ref[...], w_ref[...], preferred_element_type=jnp.float32)
    h = h + b_ref[...]
    o_ref[0] = h[:, :FH] * d_ref[...]
    o_ref[1] = h[:, FH:] * d_ref[...]


def _relu_halves(p_ref, hs_ref, d_ref):
    tl = jnp.maximum((p_ref[0] + hs_ref[0]) * d_ref[...], 0.0)
    th = jnp.maximum((p_ref[1] + hs_ref[1]) * d_ref[...], 0.0)
    return tl, th


def _mm2_body(p_ref, hs_ref, d_ref, w_ref, b_ref, o_ref):
    tl, th = _relu_halves(p_ref, hs_ref, d_ref)
    h = jnp.dot(tl, w_ref[:FH], preferred_element_type=jnp.float32)
    h = h + jnp.dot(th, w_ref[FH:], preferred_element_type=jnp.float32)
    h = h + b_ref[...]
    o_ref[0] = h[:, :FH] * d_ref[...]
    o_ref[1] = h[:, FH:] * d_ref[...]


def _finlog_body(p_ref, hs_ref, d_ref, wh_ref, bh_ref, o_ref):
    tl, th = _relu_halves(p_ref, hs_ref, d_ref)
    logits = jnp.dot(tl, wh_ref[:FH], preferred_element_type=jnp.float32)
    logits = logits + jnp.dot(th, wh_ref[FH:],
                              preferred_element_type=jnp.float32)
    o_ref[...] = logits + bh_ref[...]


def _smax_body(l_ref, o_ref):
    logits = l_ref[...]
    m = jnp.max(logits)
    e = jnp.exp(logits - m)
    o_ref[...] = e / jnp.sum(e)


def kernel(x, ei, W1, b1, W2, b2, Wh, bh):
    ei = ei.astype(jnp.int32)
    ei_msg = ei.reshape(2 * E)
    ei_deg = ei.reshape(2 * NW, DNCH, CB)

    degp = _deg_kernel(ei_deg)
    deg = degp[0, :N] + degp[1, :N] + 1.0
    dinvb = jnp.broadcast_to(lax.rsqrt(deg)[:, None], (N, FH))

    full = lambda shape: pl.BlockSpec(shape, lambda i: (0,) * len(shape))
    rows2 = pl.BlockSpec((RB, F), lambda i: (i, 0))
    rows_h = pl.BlockSpec((RB, FH), lambda i: (i, 0))
    rows3 = pl.BlockSpec((2, RB, FH), lambda i: (0, i, 0))

    hs1 = pl.pallas_call(
        _mm1_body,
        grid=(NRB,),
        in_specs=[rows2, full((F, F)), full((1, F)), rows_h],
        out_specs=rows3,
        out_shape=jax.ShapeDtypeStruct((2, N, FH), jnp.float32),
    )(x, W1, b1.reshape(1, F), dinvb)

    p1 = _msg_kernel(hs1, ei_msg)

    hs2 = pl.pallas_call(
        _mm2_body,
        grid=(NRB,),
        in_specs=[rows3, rows3, rows_h, full((F, F)), full((1, F))],
        out_specs=rows3,
        out_shape=jax.ShapeDtypeStruct((2, N, FH), jnp.float32),
    )(p1, hs1, dinvb, W2, b2.reshape(1, F))

    p2 = _msg_kernel(hs2, ei_msg)

    logits = pl.pallas_call(
        _finlog_body,
        grid=(NRB,),
        in_specs=[rows3, rows3, rows_h, full((F, 1)), full((1, 1))],
        out_specs=pl.BlockSpec((RB, 1), lambda i: (i, 0)),
        out_shape=jax.ShapeDtypeStruct((N, 1), jnp.float32),
    )(p2, hs2, dinvb, Wh, bh.reshape(1, 1))

    w = pl.pallas_call(
        _smax_body,
        out_shape=jax.ShapeDtypeStruct((N, 1), jnp.float32),
    )(logits)

    return w[:, 0]
